# baseline (speedup 1.0000x reference)
import functools

import jax
import jax.numpy as jnp
from jax import lax
from jax.experimental import pallas as pl
from jax.experimental.pallas import tpu as pltpu
from jax.experimental.pallas import tpu_sc as plsc

_TEMP = 0.07
_EPS = 1e-12

_NC = 2
_NS = 16
_NW = _NC * _NS
_L = 16

_S_SC = 16384
_CHUNK = 8192
_G = 2


def _sc_count_body(x_ref, prb_ref, out_ref, xv, prbv, cntv, *,
                   batch, num_p, r_per_tile, slab):
    wid = lax.axis_index("s") * _NC + lax.axis_index("c")
    base = wid * r_per_tile
    pltpu.sync_copy(prb_ref, prbv)
    zero16 = jnp.zeros((_L,), jnp.float32)
    for b in range(batch):
        cnts = [zero16] * num_p
        for s0 in range(0, r_per_tile, slab):
            pltpu.sync_copy(x_ref.at[b, :, pl.ds(base + s0, slab)], xv)
            for q in range(slab // (_L * _G)):
                def cbody(c, accs, _q=q):
                    xvec = [xv[c, pl.ds(_q * _L * _G + g * _L, _L)]
                            for g in range(_G)]
                    out = [None] * (num_p * _G)
                    for p in range(num_p):
                        sv = prbv[c, pl.ds(p * _L, _L)]
                        for g in range(_G):
                            out[g * num_p + p] = accs[g * num_p + p] + xvec[g] * sv
                    return tuple(out)

                accs = lax.fori_loop(
                    0, 128, cbody, tuple([zero16] * (num_p * _G)))
                for g in range(_G):
                    agp = accs[g * num_p:(g + 1) * num_p]
                    m = agp[0]
                    for p in range(1, num_p):
                        m = jnp.maximum(m, agp[p])
                    idx = jnp.full((_L,), num_p, jnp.int32)
                    for p in reversed(range(num_p)):
                        idx = jnp.where(agp[p] == m,
                                        jnp.int32(p), idx)
                    for p in range(num_p):
                        cnts[p] = cnts[p] + jnp.where(
                            idx == p, jnp.float32(1.0), jnp.float32(0.0))
        for p in range(16):
            val = cnts[p] if p < num_p else zero16
            cntv[pl.ds((b * 16 + p) * _L, _L)] = val
    pltpu.sync_copy(cntv, out_ref.at[wid])


def _tc_count_kernel(x_ref, pr_ref, cnt_ref, *, num_p, batch):
    pr = pr_ref[...]
    for b in range(batch):
        xb = x_ref[b]
        sim = jax.lax.dot_general(pr, xb, (((1,), (0,)), ((), ())),
                                  preferred_element_type=jnp.float32)
        row = jax.lax.broadcasted_iota(jnp.int32, sim.shape, 0)
        sim = jnp.where(row < num_p, sim, -jnp.inf)
        mx = jnp.max(sim, axis=0, keepdims=True)
        idx = jnp.min(jnp.where(sim == mx, row, num_p), axis=0, keepdims=True)
        onehot = (row == idx).astype(jnp.float32)
        cnt_ref[0, b] = jnp.sum(onehot, axis=1, keepdims=True)


def _loss_kernel(cntA_ref, cntSC_ref, prT_ref, q0_ref, q1_ref, out_ref,
                 *, num_q, batch):
    countsA = jnp.sum(cntA_ref[...], axis=0)
    sumsSC = jnp.sum(jnp.sum(cntSC_ref[...], axis=0), axis=1, keepdims=True)
    prT = prT_ref[...]
    q0 = q0_ref[...]
    q1 = q1_ref[...]
    rowq = jax.lax.broadcasted_iota(jnp.int32, (q0.shape[0], 1), 0)
    validq = rowq < num_q
    q0n = q0 / jnp.maximum(
        jnp.sqrt(jnp.sum(q0 * q0, axis=1, keepdims=True)), _EPS)
    q1n = q1 / jnp.maximum(
        jnp.sqrt(jnp.sum(q1 * q1, axis=1, keepdims=True)), _EPS)
    total = jnp.zeros((1, 1), jnp.float32)
    for b in range(batch):
        colb = sumsSC[b * 16:(b + 1) * 16, :]
        cnt = countsA[b] + colb
        pool = jax.lax.dot_general(prT, cnt, (((1,), (0,)), ((), ())),
                                   preferred_element_type=jnp.float32)
        n = pool / jnp.maximum(
            jnp.sqrt(jnp.sum(pool * pool, axis=0, keepdims=True)), _EPS)
        s_neg = jax.lax.dot_general(q0n, n, (((1,), (0,)), ((), ())),
                                    preferred_element_type=jnp.float32)
        logit_neg = jnp.where(validq, s_neg / _TEMP, -jnp.inf)
        m = jnp.max(logit_neg, axis=0, keepdims=True)
        eln = jnp.where(validq, jnp.exp(logit_neg - m), 0.0)
        l_neg = jnp.sum(eln, axis=0, keepdims=True)
        s_pos = jax.lax.dot_general(q1n, n, (((1,), (0,)), ((), ())),
                                    preferred_element_type=jnp.float32)
        logit_pos = s_pos / _TEMP - m
        elp = jnp.exp(logit_pos)
        terms = -(logit_pos - jnp.log(jnp.maximum(l_neg + elp, 1e-4)))
        loss_b = jnp.sum(jnp.where(validq, terms, 0.0),
                         axis=0, keepdims=True) / num_q
        total = total + loss_b
    out_ref[...] = total / batch


def kernel(x, label, prototypes, queue0, queue1):
    del label
    B, C = x.shape[0], x.shape[1]
    S = x.shape[2] * x.shape[3] * x.shape[4]
    P = prototypes.shape[0]
    Q = queue0.shape[0]
    PPAD = 16
    QPAD = 16
    nsteps = (S - _S_SC) // _CHUNK
    r_per_tile = _S_SC // _NW
    slab = 512 if r_per_tile % 512 == 0 else 256

    x2 = x.reshape(B, C, S)
    pr = jnp.zeros((PPAD, C), jnp.float32).at[:P].set(
        prototypes.reshape(P, C).astype(jnp.float32))
    prT = pr.T
    q0 = jnp.zeros((QPAD, C), jnp.float32).at[:Q].set(
        queue0.astype(jnp.float32))
    q1 = jnp.zeros((QPAD, C), jnp.float32).at[:Q].set(
        queue1.astype(jnp.float32))

    pr_bcast = jnp.broadcast_to(
        prototypes.reshape(P, C).astype(jnp.float32).T[:, :, None],
        (C, P, _L)).reshape(C, P * _L)

    mesh = plsc.VectorSubcoreMesh(core_axis_name="c", subcore_axis_name="s")
    sc_count = pl.kernel(
        functools.partial(_sc_count_body, batch=B, num_p=P,
                          r_per_tile=r_per_tile, slab=slab),
        mesh=mesh,
        out_type=jax.ShapeDtypeStruct((_NW, B * PPAD * _L), jnp.float32),
        scratch_types=[
            pltpu.VMEM((C, slab), jnp.float32),
            pltpu.VMEM((C, P * _L), jnp.float32),
            pltpu.VMEM((B * PPAD * _L,), jnp.float32),
        ],
    )
    cnt_sc = sc_count(x2, pr_bcast).reshape(_NW, B * PPAD, _L)

    cnt_tc = pl.pallas_call(
        functools.partial(_tc_count_kernel, num_p=P, batch=B),
        grid=(nsteps,),
        in_specs=[
            pl.BlockSpec((B, C, _CHUNK),
                         lambda j: (0, 0, j + _S_SC // _CHUNK)),
            pl.BlockSpec((PPAD, C), lambda j: (0, 0)),
        ],
        out_specs=pl.BlockSpec((1, B, PPAD, 1), lambda j: (j, 0, 0, 0)),
        out_shape=jax.ShapeDtypeStruct((nsteps, B, PPAD, 1), jnp.float32),
        compiler_params=pltpu.CompilerParams(
            dimension_semantics=("arbitrary",)),
    )(x2, pr)

    out = pl.pallas_call(
        functools.partial(_loss_kernel, num_q=Q, batch=B),
        out_shape=jax.ShapeDtypeStruct((1, 1), jnp.float32),
    )(cnt_tc, cnt_sc, prT, q0, q1)
    return out.reshape(1)


# speedup vs baseline: 1.8745x; 1.8745x over previous
import functools

import jax
import jax.numpy as jnp
from jax.experimental import pallas as pl
from jax.experimental.pallas import tpu as pltpu

_TEMP = 0.07
_EPS = 1e-12
_ROWS = 4096


def _count_kernel(x_ref, pr_ref, cnt_ref, *, num_p):
    xb = x_ref[0]
    pr = pr_ref[...]
    sim = jax.lax.dot_general(xb, pr, (((1,), (1,)), ((), ())),
                              preferred_element_type=jnp.float32)
    lane = jax.lax.broadcasted_iota(jnp.int32, sim.shape, 1)
    sim = jnp.where(lane < num_p, sim, -jnp.inf)
    mx = jnp.max(sim, axis=1, keepdims=True)
    idx = jnp.min(jnp.where(sim == mx, lane, num_p), axis=1, keepdims=True)
    onehot = (lane == idx).astype(jnp.float32)
    cnt_ref[0, 0] = jnp.sum(onehot, axis=0, keepdims=True)


def _loss_kernel(cnt_ref, pr_ref, q0_ref, q1_ref, out_ref, *, num_q, batch):
    cnts = jnp.sum(cnt_ref[...], axis=1)
    pr = pr_ref[...]
    q0 = q0_ref[...]
    q1 = q1_ref[...]
    laneq = jax.lax.broadcasted_iota(jnp.int32, (1, q0.shape[0]), 1)
    validq = laneq < num_q
    q0n = q0 / jnp.maximum(
        jnp.sqrt(jnp.sum(q0 * q0, axis=1, keepdims=True)), _EPS)
    q1n = q1 / jnp.maximum(
        jnp.sqrt(jnp.sum(q1 * q1, axis=1, keepdims=True)), _EPS)
    total = jnp.zeros((1, 1), jnp.float32)
    for b in range(batch):
        cntrow = cnts[b]
        pool = jax.lax.dot_general(cntrow, pr, (((1,), (0,)), ((), ())),
                                   preferred_element_type=jnp.float32)
        n = pool / jnp.maximum(
            jnp.sqrt(jnp.sum(pool * pool, axis=1, keepdims=True)), _EPS)
        s_neg = jax.lax.dot_general(n, q0n, (((1,), (1,)), ((), ())),
                                    preferred_element_type=jnp.float32)
        logit_neg = jnp.where(validq, s_neg / _TEMP, -jnp.inf)
        m = jnp.max(logit_neg, axis=1, keepdims=True)
        eln = jnp.where(validq, jnp.exp(logit_neg - m), 0.0)
        l_neg = jnp.sum(eln, axis=1, keepdims=True)
        s_pos = jax.lax.dot_general(n, q1n, (((1,), (1,)), ((), ())),
                                    preferred_element_type=jnp.float32)
        logit_pos = s_pos / _TEMP - m
        elp = jnp.exp(logit_pos)
        terms = -(logit_pos - jnp.log(jnp.maximum(l_neg + elp, 1e-4)))
        loss_b = jnp.sum(jnp.where(validq, terms, 0.0),
                         axis=1, keepdims=True) / num_q
        total = total + loss_b
    out_ref[...] = total / batch


def kernel(x, label, prototypes, queue0, queue1):
    del label
    B, C = x.shape[0], x.shape[1]
    S = x.shape[2] * x.shape[3] * x.shape[4]
    P = prototypes.shape[0]
    Q = queue0.shape[0]
    PPAD = 16
    QPAD = 16
    nsteps = S // _ROWS

    x3 = x.transpose(0, 2, 3, 4, 1).reshape(B, S, C)
    pr = jnp.zeros((PPAD, C), jnp.float32).at[:P].set(
        prototypes.reshape(P, C).astype(jnp.float32))
    q0 = jnp.zeros((QPAD, C), jnp.float32).at[:Q].set(
        queue0.astype(jnp.float32))
    q1 = jnp.zeros((QPAD, C), jnp.float32).at[:Q].set(
        queue1.astype(jnp.float32))

    cnt = pl.pallas_call(
        functools.partial(_count_kernel, num_p=P),
        grid=(B, nsteps),
        in_specs=[
            pl.BlockSpec((1, _ROWS, C), lambda b, j: (b, j, 0)),
            pl.BlockSpec((PPAD, C), lambda b, j: (0, 0)),
        ],
        out_specs=pl.BlockSpec((1, 1, 1, PPAD), lambda b, j: (b, j, 0, 0)),
        out_shape=jax.ShapeDtypeStruct((B, nsteps, 1, PPAD), jnp.float32),
        compiler_params=pltpu.CompilerParams(
            dimension_semantics=("arbitrary", "arbitrary")),
    )(x3, pr)

    out = pl.pallas_call(
        functools.partial(_loss_kernel, num_q=Q, batch=B),
        out_shape=jax.ShapeDtypeStruct((1, 1), jnp.float32),
    )(cnt, pr, q0, q1)
    return out.reshape(1)


# speedup vs baseline: 3.9866x; 2.1267x over previous
import functools

import jax
import jax.numpy as jnp
from jax.experimental import pallas as pl
from jax.experimental.pallas import tpu as pltpu

_TEMP = 0.07
_EPS = 1e-12
_ROWS = 8192


def _count_kernel(x_ref, pr_ref, cnt_ref, *, num_p):
    xb = x_ref[0]
    pr = pr_ref[...]
    sim = jax.lax.dot_general(pr, xb, (((1,), (1,)), ((), ())),
                              preferred_element_type=jnp.float32)
    row = jax.lax.broadcasted_iota(jnp.int32, sim.shape, 0)
    sim = jnp.where(row < num_p, sim, -jnp.inf)
    mx = jnp.max(sim, axis=0, keepdims=True)
    onehot = (sim == mx).astype(jnp.float32)
    ones = jnp.ones((sim.shape[1], 1), jnp.float32)
    cnt = jax.lax.dot_general(onehot, ones, (((1,), (0,)), ((), ())),
                              preferred_element_type=jnp.float32)
    cnt_ref[0, 0] = cnt


def _loss_kernel(cnt_ref, prT_ref, q0_ref, q1_ref, out_ref, *, num_q, batch):
    counts = jnp.sum(cnt_ref[...], axis=1)
    prT = prT_ref[...]
    q0 = q0_ref[...]
    q1 = q1_ref[...]
    rowq = jax.lax.broadcasted_iota(jnp.int32, (q0.shape[0], 1), 0)
    validq = rowq < num_q
    q0n = q0 / jnp.maximum(
        jnp.sqrt(jnp.sum(q0 * q0, axis=1, keepdims=True)), _EPS)
    q1n = q1 / jnp.maximum(
        jnp.sqrt(jnp.sum(q1 * q1, axis=1, keepdims=True)), _EPS)
    total = jnp.zeros((1, 1), jnp.float32)
    for b in range(batch):
        cnt = counts[b]
        pool = jax.lax.dot_general(prT, cnt, (((1,), (0,)), ((), ())),
                                   preferred_element_type=jnp.float32)
        n = pool / jnp.maximum(
            jnp.sqrt(jnp.sum(pool * pool, axis=0, keepdims=True)), _EPS)
        s_neg = jax.lax.dot_general(q0n, n, (((1,), (0,)), ((), ())),
                                    preferred_element_type=jnp.float32)
        logit_neg = jnp.where(validq, s_neg / _TEMP, -jnp.inf)
        m = jnp.max(logit_neg, axis=0, keepdims=True)
        eln = jnp.where(validq, jnp.exp(logit_neg - m), 0.0)
        l_neg = jnp.sum(eln, axis=0, keepdims=True)
        s_pos = jax.lax.dot_general(q1n, n, (((1,), (0,)), ((), ())),
                                    preferred_element_type=jnp.float32)
        logit_pos = s_pos / _TEMP - m
        elp = jnp.exp(logit_pos)
        terms = -(logit_pos - jnp.log(jnp.maximum(l_neg + elp, 1e-4)))
        loss_b = jnp.sum(jnp.where(validq, terms, 0.0),
                         axis=0, keepdims=True) / num_q
        total = total + loss_b
    out_ref[...] = total / batch


def kernel(x, label, prototypes, queue0, queue1):
    del label
    B, C = x.shape[0], x.shape[1]
    S = x.shape[2] * x.shape[3] * x.shape[4]
    P = prototypes.shape[0]
    Q = queue0.shape[0]
    PPAD = 16
    QPAD = 16
    nsteps = S // _ROWS

    x3 = x.transpose(0, 2, 3, 4, 1).reshape(B, S, C)
    pr = jnp.zeros((PPAD, C), jnp.float32).at[:P].set(
        prototypes.reshape(P, C).astype(jnp.float32))
    prT = pr.T
    q0 = jnp.zeros((QPAD, C), jnp.float32).at[:Q].set(
        queue0.astype(jnp.float32))
    q1 = jnp.zeros((QPAD, C), jnp.float32).at[:Q].set(
        queue1.astype(jnp.float32))

    cnt = pl.pallas_call(
        functools.partial(_count_kernel, num_p=P),
        grid=(B, nsteps),
        in_specs=[
            pl.BlockSpec((1, _ROWS, C), lambda b, j: (b, j, 0)),
            pl.BlockSpec((PPAD, C), lambda b, j: (0, 0)),
        ],
        out_specs=pl.BlockSpec((1, 1, PPAD, 1), lambda b, j: (b, j, 0, 0)),
        out_shape=jax.ShapeDtypeStruct((B, nsteps, PPAD, 1), jnp.float32),
        compiler_params=pltpu.CompilerParams(
            dimension_semantics=("arbitrary", "arbitrary")),
    )(x3, pr)

    out = pl.pallas_call(
        functools.partial(_loss_kernel, num_q=Q, batch=B),
        out_shape=jax.ShapeDtypeStruct((1, 1), jnp.float32),
    )(cnt, prT, q0, q1)
    return out.reshape(1)


# speedup vs baseline: 4.1701x; 1.0460x over previous
import functools

import jax
import jax.numpy as jnp
from jax.experimental import pallas as pl
from jax.experimental.pallas import tpu as pltpu

_TEMP = 0.07
_EPS = 1e-12
_ROWS = 8192


def _fused_kernel(x_ref, pr_ref, prT_ref, q0_ref, q1_ref, out_ref, acc_ref,
                  *, num_p, num_q, batch, nsteps):
    b = pl.program_id(0)
    j = pl.program_id(1)

    @pl.when((b == 0) & (j == 0))
    def _init():
        acc_ref[...] = jnp.zeros_like(acc_ref)

    xb = x_ref[0]
    pr = pr_ref[...]
    sim = jax.lax.dot_general(pr, xb, (((1,), (1,)), ((), ())),
                              preferred_element_type=jnp.float32)
    row = jax.lax.broadcasted_iota(jnp.int32, sim.shape, 0)
    sim = jnp.where(row < num_p, sim, -jnp.inf)
    mx = jnp.max(sim, axis=0, keepdims=True)
    onehot = (sim == mx).astype(jnp.float32)
    ones = jnp.ones((sim.shape[1], 1), jnp.float32)
    cnt = jax.lax.dot_general(onehot, ones, (((1,), (0,)), ((), ())),
                              preferred_element_type=jnp.float32)
    for bb in range(batch):
        @pl.when(b == bb)
        def _acc(bb=bb):
            acc_ref[bb] += cnt

    @pl.when((b == batch - 1) & (j == nsteps - 1))
    def _loss():
        prT = prT_ref[...]
        q0 = q0_ref[...]
        q1 = q1_ref[...]
        rowq = jax.lax.broadcasted_iota(jnp.int32, (q0.shape[0], 1), 0)
        validq = rowq < num_q
        q0n = q0 / jnp.maximum(
            jnp.sqrt(jnp.sum(q0 * q0, axis=1, keepdims=True)), _EPS)
        q1n = q1 / jnp.maximum(
            jnp.sqrt(jnp.sum(q1 * q1, axis=1, keepdims=True)), _EPS)
        total = jnp.zeros((1, 1), jnp.float32)
        for bb in range(batch):
            cntb = acc_ref[bb]
            pool = jax.lax.dot_general(prT, cntb, (((1,), (0,)), ((), ())),
                                       preferred_element_type=jnp.float32)
            n = pool / jnp.maximum(
                jnp.sqrt(jnp.sum(pool * pool, axis=0, keepdims=True)), _EPS)
            s_neg = jax.lax.dot_general(q0n, n, (((1,), (0,)), ((), ())),
                                        preferred_element_type=jnp.float32)
            logit_neg = jnp.where(validq, s_neg / _TEMP, -jnp.inf)
            m = jnp.max(logit_neg, axis=0, keepdims=True)
            eln = jnp.where(validq, jnp.exp(logit_neg - m), 0.0)
            l_neg = jnp.sum(eln, axis=0, keepdims=True)
            s_pos = jax.lax.dot_general(q1n, n, (((1,), (0,)), ((), ())),
                                        preferred_element_type=jnp.float32)
            logit_pos = s_pos / _TEMP - m
            elp = jnp.exp(logit_pos)
            terms = -(logit_pos - jnp.log(jnp.maximum(l_neg + elp, 1e-4)))
            loss_b = jnp.sum(jnp.where(validq, terms, 0.0),
                             axis=0, keepdims=True) / num_q
            total = total + loss_b
        out_ref[...] = total / batch


def kernel(x, label, prototypes, queue0, queue1):
    del label
    B, C = x.shape[0], x.shape[1]
    S = x.shape[2] * x.shape[3] * x.shape[4]
    P = prototypes.shape[0]
    Q = queue0.shape[0]
    PPAD = 16
    QPAD = 16
    nsteps = S // _ROWS

    x3 = x.transpose(0, 2, 3, 4, 1).reshape(B, S, C)
    pr = jnp.zeros((PPAD, C), jnp.float32).at[:P].set(
        prototypes.reshape(P, C).astype(jnp.float32))
    prT = pr.T
    q0 = jnp.zeros((QPAD, C), jnp.float32).at[:Q].set(
        queue0.astype(jnp.float32))
    q1 = jnp.zeros((QPAD, C), jnp.float32).at[:Q].set(
        queue1.astype(jnp.float32))

    out = pl.pallas_call(
        functools.partial(_fused_kernel, num_p=P, num_q=Q, batch=B,
                          nsteps=nsteps),
        grid=(B, nsteps),
        in_specs=[
            pl.BlockSpec((1, _ROWS, C), lambda b, j: (b, j, 0)),
            pl.BlockSpec((PPAD, C), lambda b, j: (0, 0)),
            pl.BlockSpec((C, PPAD), lambda b, j: (0, 0)),
            pl.BlockSpec((QPAD, C), lambda b, j: (0, 0)),
            pl.BlockSpec((QPAD, C), lambda b, j: (0, 0)),
        ],
        out_specs=pl.BlockSpec((1, 1), lambda b, j: (0, 0)),
        out_shape=jax.ShapeDtypeStruct((1, 1), jnp.float32),
        scratch_shapes=[pltpu.VMEM((B, 16, 1), jnp.float32)],
        compiler_params=pltpu.CompilerParams(
            dimension_semantics=("arbitrary", "arbitrary")),
    )(x3, pr, prT, q0, q1)
    return out.reshape(1)


# speedup vs baseline: 4.6610x; 1.1177x over previous
import functools

import jax
import jax.numpy as jnp
from jax.experimental import pallas as pl
from jax.experimental.pallas import tpu as pltpu

_TEMP = 0.07
_EPS = 1e-12
_ROWS = 16384


def _fused_kernel(x_ref, pr_ref, prT_ref, q0_ref, q1_ref, out_ref, acc_ref,
                  *, num_p, num_q, batch, nsteps):
    b = pl.program_id(0)
    j = pl.program_id(1)

    @pl.when((b == 0) & (j == 0))
    def _init():
        acc_ref[...] = jnp.zeros_like(acc_ref)

    xb = x_ref[0]
    pr = pr_ref[...]
    sim = jax.lax.dot_general(pr, xb, (((1,), (1,)), ((), ())),
                              preferred_element_type=jnp.float32)
    row = jax.lax.broadcasted_iota(jnp.int32, sim.shape, 0)
    sim = jnp.where(row < num_p, sim, -jnp.inf)
    mx = jnp.max(sim, axis=0, keepdims=True)
    onehot = (sim == mx).astype(jnp.float32)
    ones = jnp.ones((sim.shape[1], 1), jnp.float32)
    cnt = jax.lax.dot_general(onehot, ones, (((1,), (0,)), ((), ())),
                              preferred_element_type=jnp.float32)
    for bb in range(batch):
        @pl.when(b == bb)
        def _acc(bb=bb):
            acc_ref[bb] += cnt

    @pl.when((b == batch - 1) & (j == nsteps - 1))
    def _loss():
        prT = prT_ref[...]
        q0 = q0_ref[...]
        q1 = q1_ref[...]
        rowq = jax.lax.broadcasted_iota(jnp.int32, (q0.shape[0], 1), 0)
        validq = rowq < num_q
        q0n = q0 / jnp.maximum(
            jnp.sqrt(jnp.sum(q0 * q0, axis=1, keepdims=True)), _EPS)
        q1n = q1 / jnp.maximum(
            jnp.sqrt(jnp.sum(q1 * q1, axis=1, keepdims=True)), _EPS)
        total = jnp.zeros((1, 1), jnp.float32)
        for bb in range(batch):
            cntb = acc_ref[bb]
            pool = jax.lax.dot_general(prT, cntb, (((1,), (0,)), ((), ())),
                                       preferred_element_type=jnp.float32)
            n = pool / jnp.maximum(
                jnp.sqrt(jnp.sum(pool * pool, axis=0, keepdims=True)), _EPS)
            s_neg = jax.lax.dot_general(q0n, n, (((1,), (0,)), ((), ())),
                                        preferred_element_type=jnp.float32)
            logit_neg = jnp.where(validq, s_neg / _TEMP, -jnp.inf)
            m = jnp.max(logit_neg, axis=0, keepdims=True)
            eln = jnp.where(validq, jnp.exp(logit_neg - m), 0.0)
            l_neg = jnp.sum(eln, axis=0, keepdims=True)
            s_pos = jax.lax.dot_general(q1n, n, (((1,), (0,)), ((), ())),
                                        preferred_element_type=jnp.float32)
            logit_pos = s_pos / _TEMP - m
            elp = jnp.exp(logit_pos)
            terms = -(logit_pos - jnp.log(jnp.maximum(l_neg + elp, 1e-4)))
            loss_b = jnp.sum(jnp.where(validq, terms, 0.0),
                             axis=0, keepdims=True) / num_q
            total = total + loss_b
        out_ref[...] = total / batch


def kernel(x, label, prototypes, queue0, queue1):
    del label
    B, C = x.shape[0], x.shape[1]
    S = x.shape[2] * x.shape[3] * x.shape[4]
    P = prototypes.shape[0]
    Q = queue0.shape[0]
    PPAD = 16
    QPAD = 16
    nsteps = S // _ROWS

    x3 = x.transpose(0, 2, 3, 4, 1).reshape(B, S, C)
    pr = jnp.zeros((PPAD, C), jnp.float32).at[:P].set(
        prototypes.reshape(P, C).astype(jnp.float32))
    prT = pr.T
    q0 = jnp.zeros((QPAD, C), jnp.float32).at[:Q].set(
        queue0.astype(jnp.float32))
    q1 = jnp.zeros((QPAD, C), jnp.float32).at[:Q].set(
        queue1.astype(jnp.float32))

    out = pl.pallas_call(
        functools.partial(_fused_kernel, num_p=P, num_q=Q, batch=B,
                          nsteps=nsteps),
        grid=(B, nsteps),
        in_specs=[
            pl.BlockSpec((1, _ROWS, C), lambda b, j: (b, j, 0)),
            pl.BlockSpec((PPAD, C), lambda b, j: (0, 0)),
            pl.BlockSpec((C, PPAD), lambda b, j: (0, 0)),
            pl.BlockSpec((QPAD, C), lambda b, j: (0, 0)),
            pl.BlockSpec((QPAD, C), lambda b, j: (0, 0)),
        ],
        out_specs=pl.BlockSpec((1, 1), lambda b, j: (0, 0)),
        out_shape=jax.ShapeDtypeStruct((1, 1), jnp.float32),
        scratch_shapes=[pltpu.VMEM((B, 16, 1), jnp.float32)],
        compiler_params=pltpu.CompilerParams(
            dimension_semantics=("arbitrary", "arbitrary")),
    )(x3, pr, prT, q0, q1)
    return out.reshape(1)
